# baseline (speedup 1.0000x reference)
import jax
import jax.numpy as jnp
from jax import lax
from jax.experimental import pallas as pl
from jax.experimental.pallas import tpu as pltpu
from jax.experimental.pallas import tpu_sc as plsc

N_NODES = 10000
D = 128
NUM_RELS = 8
ALPHA = 0.5
SQRT_A = float(ALPHA ** 0.5)
SQRT_1MA = float((1.0 - ALPHA) ** 0.5)

NC = 2
NS = 16
NW = NC * NS
B = 128
NB = 80
NH = NB // 2
E_PAD = NW * NB * B
ACC_ROWS = 10112
ROWS_PER_SUB = ACC_ROWS // NS

NP = 10240
BN = 2048
TABLE_ROWS = NUM_RELS * NP


def _rel_transform_body(x_ref, w_ref, out_ref):
    out_ref[...] = jnp.dot(x_ref[...], w_ref[0],
                           preferred_element_type=jnp.float32)


def _rel_transform(x_pad, weight):
    nb = NP // BN
    return pl.pallas_call(
        _rel_transform_body,
        grid=(nb, NUM_RELS),
        in_specs=[
            pl.BlockSpec((BN, D), lambda n, r: (n, 0)),
            pl.BlockSpec((1, D, D), lambda n, r: (r, 0, 0)),
        ],
        out_specs=pl.BlockSpec((BN, D), lambda n, r: (r * nb + n, 0)),
        out_shape=jax.ShapeDtypeStruct((TABLE_ROWS, D), jnp.float32),
    )(x_pad, weight)


def _sc_body(table, src_idx_hbm, dst_idx_hbm, partial,
             src_idx, dst_idx, rows0, rows1, acc, gsem0, gsem1):
    c = lax.axis_index("c")
    s = lax.axis_index("s")
    w = (1 - c) * NS + s
    bufs = (rows0, rows1)
    sems = (gsem0, gsem1)

    zero = jnp.zeros((16,), jnp.float32)

    def _z(i, carry):
        rows0[i // 8, pl.ds((i % 8) * 16, 16)] = zero
        return carry

    lax.fori_loop(0, B * (D // 16), _z, 0)

    base = s * ROWS_PER_SUB
    full, tail = divmod(ROWS_PER_SUB, B)
    chunks = [B] * full + ([tail] if tail else [])
    off = 0
    for sz in chunks:
        pltpu.sync_copy(rows0.at[pl.ds(0, sz)],
                        acc.at[pl.ds(base + off, sz)])
        off += sz

    plsc.subcore_barrier()

    def _gather_start(j, b):
        pltpu.async_copy(table.at[src_idx.at[j]], bufs[b], sems[b])

    def _gather_wait(j, b):
        pltpu.make_async_copy(table.at[src_idx.at[j]], bufs[b], sems[b]).wait()

    for h in range(2):
        pltpu.sync_copy(src_idx_hbm.at[w, pl.ds(h * NH, NH)], src_idx)
        pltpu.sync_copy(dst_idx_hbm.at[w, pl.ds(h * NH, NH)], dst_idx)

        _gather_start(0, 0)
        _gather_start(1, 1)

        def _step(g, carry):
            for b in range(2):
                j = 2 * g + b
                _gather_wait(j, b)
                pltpu.sync_copy(bufs[b], acc.at[dst_idx.at[j]], add=True)
                _gather_start(j + 2, b)
            return carry

        lax.fori_loop(0, NH // 2 - 1, _step, 0)
        for b in range(2):
            j = NH - 2 + b
            _gather_wait(j, b)
            pltpu.sync_copy(bufs[b], acc.at[dst_idx.at[j]], add=True)

    plsc.subcore_barrier()

    off = 0
    for k, sz in enumerate(chunks):
        buf = bufs[k % 2]
        pltpu.sync_copy(acc.at[pl.ds(base + off, sz)], buf.at[pl.ds(0, sz)])
        pltpu.sync_copy(buf.at[pl.ds(0, sz)],
                        partial.at[c, pl.ds(base + off, sz)])
        off += sz


def _sc_aggregate(table, src_idx, dst_idx):
    mesh = plsc.VectorSubcoreMesh(core_axis_name="c", subcore_axis_name="s")
    kfn = pl.kernel(
        _sc_body,
        out_type=jax.ShapeDtypeStruct((NC, ACC_ROWS, D), jnp.float32),
        mesh=mesh,
        scratch_types=[
            pltpu.VMEM((NH, B), jnp.int32),
            pltpu.VMEM((NH, B), jnp.int32),
            pltpu.VMEM((B, D), jnp.float32),
            pltpu.VMEM((B, D), jnp.float32),
            pltpu.VMEM_SHARED((ACC_ROWS, D), jnp.float32),
            pltpu.SemaphoreType.DMA,
            pltpu.SemaphoreType.DMA,
        ],
    )
    return kfn(table, src_idx, dst_idx)


def _combine_body(p_ref, x_ref, lw_ref, b_ref, out_ref):
    agg = (p_ref[0] + p_ref[1]) * SQRT_A
    loop = jnp.dot(x_ref[...], lw_ref[...],
                   preferred_element_type=jnp.float32)
    out_ref[...] = agg + b_ref[...] + loop * SQRT_1MA


def _combine(partial, x, loop_weight, h_bias2d):
    bn = 2000
    nb = N_NODES // bn
    return pl.pallas_call(
        _combine_body,
        grid=(nb,),
        in_specs=[
            pl.BlockSpec((NC, bn, D), lambda n: (0, n, 0)),
            pl.BlockSpec((bn, D), lambda n: (n, 0)),
            pl.BlockSpec((D, D), lambda n: (0, 0)),
            pl.BlockSpec((1, D), lambda n: (0, 0)),
        ],
        out_specs=pl.BlockSpec((bn, D), lambda n: (n, 0)),
        out_shape=jax.ShapeDtypeStruct((N_NODES, D), jnp.float32),
    )(partial, x, loop_weight, h_bias2d)


def kernel(edge_index, x, edge_type, weight, h_bias, loop_weight):
    src = edge_index[0].astype(jnp.int32)
    dst = edge_index[1].astype(jnp.int32)
    et = edge_type.astype(jnp.int32)
    n_edges = src.shape[0]
    pad = E_PAD - n_edges
    flat_src = jnp.pad(et * NP + src, (0, pad), constant_values=N_NODES)
    dst_pad = jnp.pad(dst, (0, pad))
    src_idx = flat_src.reshape(NW, NB, B)
    dst_idx = dst_pad.reshape(NW, NB, B)

    x_pad = jnp.pad(x, ((0, NP - N_NODES), (0, 0)))
    table = _rel_transform(x_pad, weight)
    partial = _sc_aggregate(table, src_idx, dst_idx)
    return _combine(partial, x, loop_weight, h_bias.reshape(1, D))


# speedup vs baseline: 2.5925x; 2.5925x over previous
import jax
import jax.numpy as jnp
from jax import lax
from jax.experimental import pallas as pl
from jax.experimental.pallas import tpu as pltpu
from jax.experimental.pallas import tpu_sc as plsc

N_NODES = 10000
D = 128
NUM_RELS = 8
ALPHA = 0.5
SQRT_A = float(ALPHA ** 0.5)
SQRT_1MA = float((1.0 - ALPHA) ** 0.5)

NC = 2
NS = 16
NW = NC * NS
B = 128
NB = 80
NH = NB // 2
E_PAD = NW * NB * B
ACC_ROWS = 10112
ROWS_PER_SUB = ACC_ROWS // NS

NP = 10240
BN = 2048
TABLE_ROWS = NUM_RELS * NP


def _rel_transform_body(x_ref, w_ref, out_ref):
    out_ref[...] = jnp.dot(x_ref[...], w_ref[0],
                           preferred_element_type=jnp.float32)


def _rel_transform(x_pad, weight):
    nb = NP // BN
    return pl.pallas_call(
        _rel_transform_body,
        grid=(nb, NUM_RELS),
        in_specs=[
            pl.BlockSpec((BN, D), lambda n, r: (n, 0)),
            pl.BlockSpec((1, D, D), lambda n, r: (r, 0, 0)),
        ],
        out_specs=pl.BlockSpec((BN, D), lambda n, r: (r * nb + n, 0)),
        out_shape=jax.ShapeDtypeStruct((TABLE_ROWS, D), jnp.float32),
    )(x_pad, weight)


def _sc_body(table, src_idx_hbm, dst_idx_hbm, partial,
             src_idx, dst_idx, rows0, rows1, acc, gsem0, gsem1):
    c = lax.axis_index("c")
    s = lax.axis_index("s")
    w = c * NS + s
    bufs = (rows0, rows1)
    sems = (gsem0, gsem1)

    zero = jnp.zeros((16,), jnp.float32)

    def _z(i, carry):
        rows0[i // 8, pl.ds((i % 8) * 16, 16)] = zero
        return carry

    lax.fori_loop(0, B * (D // 16), _z, 0)

    base = s * ROWS_PER_SUB
    full, tail = divmod(ROWS_PER_SUB, B)
    chunks = [B] * full + ([tail] if tail else [])
    off = 0
    for sz in chunks:
        pltpu.sync_copy(rows0.at[pl.ds(0, sz)],
                        acc.at[pl.ds(base + off, sz)])
        off += sz

    plsc.subcore_barrier()

    def _gather_start(j, b):
        pltpu.async_copy(table.at[src_idx.at[j]], bufs[b], sems[b])

    def _gather_wait(j, b):
        pltpu.make_async_copy(table.at[src_idx.at[j]], bufs[b], sems[b]).wait()

    for h in range(2):
        pltpu.sync_copy(src_idx_hbm.at[w, pl.ds(h * NH, NH)], src_idx)
        pltpu.sync_copy(dst_idx_hbm.at[w, pl.ds(h * NH, NH)], dst_idx)

        _gather_start(0, 0)
        _gather_start(1, 1)

        def _step(g, carry):
            for b in range(2):
                j = 2 * g + b
                _gather_wait(j, b)
                pltpu.sync_copy(bufs[b], acc.at[dst_idx.at[j]], add=True)
                _gather_start(j + 2, b)
            return carry

        lax.fori_loop(0, NH // 2 - 1, _step, 0)
        for b in range(2):
            j = NH - 2 + b
            _gather_wait(j, b)
            pltpu.sync_copy(bufs[b], acc.at[dst_idx.at[j]], add=True)

    plsc.subcore_barrier()

    off = 0
    for k, sz in enumerate(chunks):
        buf = bufs[k % 2]
        pltpu.sync_copy(acc.at[pl.ds(base + off, sz)], buf.at[pl.ds(0, sz)])
        pltpu.sync_copy(buf.at[pl.ds(0, sz)],
                        partial.at[c, pl.ds(base + off, sz)])
        off += sz


def _sc_aggregate(table, src_idx, dst_idx):
    mesh = plsc.VectorSubcoreMesh(core_axis_name="c", subcore_axis_name="s")
    kfn = pl.kernel(
        _sc_body,
        out_type=jax.ShapeDtypeStruct((NC, ACC_ROWS, D), jnp.float32),
        mesh=mesh,
        scratch_types=[
            pltpu.VMEM((NH, B), jnp.int32),
            pltpu.VMEM((NH, B), jnp.int32),
            pltpu.VMEM((B, D), jnp.float32),
            pltpu.VMEM((B, D), jnp.float32),
            pltpu.VMEM_SHARED((ACC_ROWS, D), jnp.float32),
            pltpu.SemaphoreType.DMA,
            pltpu.SemaphoreType.DMA,
        ],
    )
    return kfn(table, src_idx, dst_idx)


def _combine_body(p_ref, x_ref, lw_ref, b_ref, out_ref):
    agg = (p_ref[0] + p_ref[1]) * SQRT_A
    loop = jnp.dot(x_ref[...], lw_ref[...],
                   preferred_element_type=jnp.float32)
    out_ref[...] = agg + b_ref[...] + loop * SQRT_1MA


def _combine(partial, x, loop_weight, h_bias2d):
    bn = 2000
    nb = N_NODES // bn
    return pl.pallas_call(
        _combine_body,
        grid=(nb,),
        in_specs=[
            pl.BlockSpec((NC, bn, D), lambda n: (0, n, 0)),
            pl.BlockSpec((bn, D), lambda n: (n, 0)),
            pl.BlockSpec((D, D), lambda n: (0, 0)),
            pl.BlockSpec((1, D), lambda n: (0, 0)),
        ],
        out_specs=pl.BlockSpec((bn, D), lambda n: (n, 0)),
        out_shape=jax.ShapeDtypeStruct((N_NODES, D), jnp.float32),
    )(partial, x, loop_weight, h_bias2d)


def kernel(edge_index, x, edge_type, weight, h_bias, loop_weight):
    src = edge_index[0].astype(jnp.int32)
    dst = edge_index[1].astype(jnp.int32)
    et = edge_type.astype(jnp.int32)
    n_edges = src.shape[0]
    pad = E_PAD - n_edges
    pad_src = N_NODES + (jnp.arange(pad, dtype=jnp.int32) % (NP - N_NODES))
    pad_dst = jnp.arange(pad, dtype=jnp.int32) % N_NODES
    flat_src = jnp.concatenate([et * NP + src, pad_src])
    dst_pad = jnp.concatenate([dst, pad_dst])
    src_idx = flat_src.reshape(NW, NB, B)
    dst_idx = dst_pad.reshape(NW, NB, B)

    x_pad = jnp.pad(x, ((0, NP - N_NODES), (0, 0)))
    table = _rel_transform(x_pad, weight)
    partial = _sc_aggregate(table, src_idx, dst_idx)
    return _combine(partial, x, loop_weight, h_bias.reshape(1, D))
